# baseline (speedup 1.0000x reference)
import functools

import jax
import jax.numpy as jnp
from jax import lax
from jax.experimental import pallas as pl
from jax.experimental.pallas import tpu as pltpu
from jax.experimental.pallas import tpu_sc as plsc

E = 64
D_MODEL = 768
D_FF = 3072
T = 2048
BT = 64
NF = 2
X_ROWS = 3072
NC = 2
NS = 16
NW = NC * NS
EPW = E // NW
L = 16
NV = T // L
C16 = 16

@functools.cache
def _sc_mesh():
    return plsc.VectorSubcoreMesh(core_axis_name="c", subcore_axis_name="s")


def _wid():
    return lax.axis_index("s") * NC + lax.axis_index("c")


def _r16(n):
    return ((n + 15) >> 4) << 4


def _sc_count_body(idx_hbm, counts_hbm, idx_v, row_v, sem):
    w = _wid()
    pltpu.async_copy(idx_hbm, idx_v, sem).wait()
    iota = lax.iota(jnp.int32, L)

    def step(i, carry):
        c = list(carry)
        off = pl.multiple_of(i * L, L)
        v = idx_v[pl.ds(off, L)]
        for j in range(EPW):
            c[j] = c[j] + jnp.sum((v == w * EPW + j).astype(jnp.int32))
        return tuple(c)

    counts = lax.fori_loop(0, NV, step, (jnp.int32(0),) * EPW)
    row = jnp.zeros((L,), jnp.int32)
    for j in range(EPW):
        row = jnp.where(iota == j, counts[j], row)
    row_v[...] = row
    pltpu.async_copy(row_v, counts_hbm.at[w], sem).wait()


def _sc_count(idx):
    return pl.kernel(
        _sc_count_body,
        out_type=jax.ShapeDtypeStruct((NW, L), jnp.int32),
        mesh=_sc_mesh(),
        compiler_params=pltpu.CompilerParams(needs_layout_passes=False),
        scratch_types=[
            pltpu.VMEM((T,), jnp.int32),
            pltpu.VMEM((L,), jnp.int32),
            pltpu.SemaphoreType.DMA,
        ],
    )(idx)


def _sc_dispatch_body(idx_hbm, gates_hbm, x_hbm, counts_hbm,
                      perm_hbm, gsort_hbm, xsort_hbm, meta_hbm,
                      idx_v, g_v, cnt_v, perm_v, gbuf, row_v, xbuf, sem):
    w = _wid()
    pltpu.async_copy(idx_hbm, idx_v, sem).wait()
    pltpu.async_copy(gates_hbm, g_v, sem).wait()
    pltpu.async_copy(counts_hbm, cnt_v, sem).wait()
    iota = lax.iota(jnp.int32, L)

    def scan_row(r, carry):
        acc, s0, s1 = carry
        crow = cnt_v[r, :]
        c0 = jnp.sum(jnp.where(iota == 0, crow, 0))
        c1 = jnp.sum(jnp.where(iota == 1, crow, 0))
        s0 = jnp.where(r == w, acc, s0)
        s1 = jnp.where(r == w, acc + _r16(c0), s1)
        return acc + _r16(c0) + _r16(c1), s0, s1

    _, s0, s1 = lax.fori_loop(0, NW, scan_row, (jnp.int32(0),) * 3)
    myrow = cnt_v[w, :]
    c0 = jnp.sum(jnp.where(iota == 0, myrow, 0))
    c1 = jnp.sum(jnp.where(iota == 1, myrow, 0))

    row = jnp.zeros((L,), jnp.int32)
    for lane, val in ((0, c0), (1, c1), (2, s0), (3, s1)):
        row = jnp.where(iota == lane, val, row)
    row_v[...] = row
    pltpu.async_copy(row_v, meta_hbm.at[w], sem).wait()

    for j, (s_j, c_j) in enumerate(((s0, c0), (s1, c1))):
        e = w * EPW + j

        def build(i, ptr):
            off = pl.multiple_of(i * L, L)
            v = idx_v[pl.ds(off, L)]
            m = v == e
            pos = ptr + plsc.cumsum(m.astype(jnp.int32)) - 1
            plsc.store_scatter(perm_v, [pos], i * L + iota, mask=m)
            return ptr + plsc.all_reduce_population_count(m)

        end = lax.fori_loop(0, NV, build,
                            jnp.full((L,), 0, jnp.int32) + s_j)
        padmask = iota < _r16(c_j) - c_j
        plsc.store_scatter(perm_v, [end + iota], jnp.full((L,), T, jnp.int32),
                           mask=padmask)

        def flush(k, carry):
            off = pl.multiple_of(s_j + k * C16, C16)
            permvec = perm_v[pl.ds(off, C16)]
            valid = k * C16 + iota < c_j
            gv = plsc.load_gather(g_v, [permvec], mask=valid)
            gbuf[...] = jnp.where(valid, gv, jnp.zeros((L,), jnp.float32))
            pltpu.async_copy(perm_v.at[pl.ds(off, C16)],
                             perm_hbm.at[pl.ds(off, C16)], sem).wait()
            pltpu.async_copy(gbuf, gsort_hbm.at[pl.ds(off, C16)], sem).wait()
            pltpu.async_copy(x_hbm.at[perm_v.at[pl.ds(off, C16)]], xbuf,
                             sem).wait()
            pltpu.async_copy(xbuf, xsort_hbm.at[pl.ds(off, C16)], sem).wait()
            return carry

        lax.fori_loop(0, _r16(c_j) // C16, flush, 0)


def _sc_dispatch(idx, gates, x_pad, counts):
    return pl.kernel(
        _sc_dispatch_body,
        out_type=(
            jax.ShapeDtypeStruct((X_ROWS,), jnp.int32),
            jax.ShapeDtypeStruct((X_ROWS,), jnp.float32),
            jax.ShapeDtypeStruct((X_ROWS, D_MODEL), jnp.float32),
            jax.ShapeDtypeStruct((NW, L), jnp.int32),
        ),
        mesh=_sc_mesh(),
        compiler_params=pltpu.CompilerParams(needs_layout_passes=False),
        scratch_types=[
            pltpu.VMEM((T,), jnp.int32),
            pltpu.VMEM((T,), jnp.float32),
            pltpu.VMEM((NW, L), jnp.int32),
            pltpu.VMEM((X_ROWS,), jnp.int32),
            pltpu.VMEM((L,), jnp.float32),
            pltpu.VMEM((L,), jnp.int32),
            pltpu.VMEM((C16, D_MODEL), jnp.float32),
            pltpu.SemaphoreType.DMA,
        ],
    )(idx, gates, x_pad, counts)


def _sc_combine_body(ysort_hbm, perm_hbm, meta_hbm, outp_hbm,
                     cnt_v, perm_all, idxbuf, ybuf, sem):
    w = _wid()
    pltpu.async_copy(meta_hbm, cnt_v, sem).wait()
    pltpu.async_copy(perm_hbm, perm_all, sem).wait()
    iota = lax.iota(jnp.int32, L)
    myrow = cnt_v[w, :]
    c0 = jnp.sum(jnp.where(iota == 0, myrow, 0))
    c1 = jnp.sum(jnp.where(iota == 1, myrow, 0))
    s0 = jnp.sum(jnp.where(iota == 2, myrow, 0))
    s1 = jnp.sum(jnp.where(iota == 3, myrow, 0))

    for s_j, c_j in ((s0, c0), (s1, c1)):
        def push(k, carry):
            off = pl.multiple_of(s_j + k * C16, C16)
            idxbuf[...] = perm_all[pl.ds(off, C16)]
            pltpu.async_copy(ysort_hbm.at[pl.ds(off, C16)], ybuf, sem).wait()
            pltpu.async_copy(ybuf, outp_hbm.at[idxbuf], sem).wait()
            return carry

        lax.fori_loop(0, _r16(c_j) // C16, push, 0)


def _sc_combine(ysort, perm, meta):
    return pl.kernel(
        _sc_combine_body,
        out_type=jax.ShapeDtypeStruct((T + 8, D_MODEL), jnp.float32),
        mesh=_sc_mesh(),
        compiler_params=pltpu.CompilerParams(needs_layout_passes=False),
        scratch_types=[
            pltpu.VMEM((NW, L), jnp.int32),
            pltpu.VMEM((X_ROWS,), jnp.int32),
            pltpu.VMEM((C16,), jnp.int32),
            pltpu.VMEM((C16, D_MODEL), jnp.float32),
            pltpu.SemaphoreType.DMA,
        ],
    )(ysort, perm, meta)


def _ffn_body(meta_ref, x_ref, g_ref, w1_ref, w2_ref, y_ref):
    e = pl.program_id(0)
    s = meta_ref[0, e]
    n = meta_ref[1, e]
    nch = (n + BT - 1) // BT

    def chunk(i, carry):
        off = pl.multiple_of(s + i * BT, 8)
        xb = x_ref[pl.ds(off, BT), :]
        h = jax.lax.dot_general(xb, w1_ref[0], (((1,), (1,)), ((), ())),
                                preferred_element_type=jnp.float32)
        h = jax.nn.gelu(h)
        y = jax.lax.dot_general(h, w2_ref[0], (((1,), (1,)), ((), ())),
                                preferred_element_type=jnp.float32)
        y = y * g_ref[pl.ds(off, BT), :]
        y_ref[pl.ds(off, BT), :] = y
        return carry

    jax.lax.fori_loop(0, nch, chunk, 0)


def _grouped_ffn(meta, x_sorted, gates_sorted, W1, W2, interpret=False):
    grid_spec = pltpu.PrefetchScalarGridSpec(
        num_scalar_prefetch=1,
        grid=(E,),
        in_specs=[
            pl.BlockSpec((X_ROWS, D_MODEL), lambda e, m: (0, 0)),
            pl.BlockSpec((X_ROWS, 1), lambda e, m: (0, 0)),
            pl.BlockSpec((1, D_FF, D_MODEL), lambda e, m: (e, 0, 0)),
            pl.BlockSpec((1, D_MODEL, D_FF), lambda e, m: (e, 0, 0)),
        ],
        out_specs=pl.BlockSpec((X_ROWS, D_MODEL), lambda e, m: (0, 0)),
    )
    return pl.pallas_call(
        _ffn_body,
        grid_spec=grid_spec,
        out_shape=jax.ShapeDtypeStruct((X_ROWS, D_MODEL), jnp.float32),
        interpret=interpret,
    )(meta, x_sorted, gates_sorted, W1, W2)


def _dispatch_jnp(x, idx, gates):
    counts = jnp.sum(idx[:, None] == jnp.arange(E)[None, :], axis=0)
    padded = (counts + 7) & ~7
    starts = jnp.concatenate([jnp.zeros((1,), jnp.int32),
                              jnp.cumsum(padded)[:-1].astype(jnp.int32)])
    tight = jnp.concatenate([jnp.zeros((1,), jnp.int32),
                             jnp.cumsum(counts)[:-1].astype(jnp.int32)])
    order = jnp.argsort(idx)
    se = idx[order]
    dest = starts[se] + (jnp.arange(T, dtype=jnp.int32) - tight[se])
    perm = jnp.full((X_ROWS,), T, jnp.int32).at[dest].set(order)
    pos = jnp.zeros((T,), jnp.int32).at[order].set(dest)
    x_pad = jnp.concatenate([x, jnp.zeros((8, D_MODEL), x.dtype)])
    g_pad = jnp.concatenate([gates, jnp.zeros((8,), gates.dtype)])
    x_sorted = x_pad[perm]
    gates_sorted = g_pad[perm]
    meta = jnp.stack([starts, counts.astype(jnp.int32)])
    return meta, x_sorted, gates_sorted, pos


def kernel(input, expert_probs, expert_indices, W1, W2):
    x = input.reshape(-1, D_MODEL)
    idx = expert_indices.reshape(-1).astype(jnp.int32)
    gates = expert_probs.reshape(-1)
    x_pad = jnp.concatenate([x, jnp.zeros((8, D_MODEL), x.dtype)])
    counts = _sc_count(idx)
    perm, gsort, xsort, meta = _sc_dispatch(idx, gates, x_pad, counts)
    meta_tc = jnp.stack([meta[:, 2:2 + EPW].reshape(E),
                         meta[:, 0:EPW].reshape(E)])
    ysort = _grouped_ffn(meta_tc, xsort, gsort[:, None], W1, W2)
    outp = _sc_combine(ysort, perm, meta)
    return outp[:T].reshape(*input.shape[:-1], D_MODEL)


# speedup vs baseline: 1.0462x; 1.0462x over previous
import functools

import jax
import jax.numpy as jnp
from jax import lax
from jax.experimental import pallas as pl
from jax.experimental.pallas import tpu as pltpu
from jax.experimental.pallas import tpu_sc as plsc

E = 64
D_MODEL = 768
D_FF = 3072
T = 2048
BT = 64
X_ROWS = 2560
NC = 2
NS = 16
NW = NC * NS
EPW = E // NW
L = 16
NV = T // L
C8 = 8

@functools.cache
def _sc_mesh():
    return plsc.VectorSubcoreMesh(core_axis_name="c", subcore_axis_name="s")


def _wid():
    return lax.axis_index("s") * NC + lax.axis_index("c")


def _r8(n):
    return ((n + 7) >> 3) << 3


def _sc_count_body(idx_hbm, counts_hbm, idx_v, row_v, sem):
    w = _wid()
    pltpu.async_copy(idx_hbm, idx_v, sem).wait()
    iota = lax.iota(jnp.int32, L)

    def step(i, carry):
        c = list(carry)
        off = pl.multiple_of(i * L, L)
        v = idx_v[pl.ds(off, L)]
        for j in range(EPW):
            c[j] = c[j] + jnp.sum((v == w * EPW + j).astype(jnp.int32))
        return tuple(c)

    counts = lax.fori_loop(0, NV, step, (jnp.int32(0),) * EPW)
    row = jnp.zeros((L,), jnp.int32)
    for j in range(EPW):
        row = jnp.where(iota == j, counts[j], row)
    row_v[...] = row
    pltpu.async_copy(row_v, counts_hbm.at[w], sem).wait()


def _sc_count(idx):
    return pl.kernel(
        _sc_count_body,
        out_type=jax.ShapeDtypeStruct((NW, L), jnp.int32),
        mesh=_sc_mesh(),
        compiler_params=pltpu.CompilerParams(needs_layout_passes=False),
        scratch_types=[
            pltpu.VMEM((T,), jnp.int32),
            pltpu.VMEM((L,), jnp.int32),
            pltpu.SemaphoreType.DMA,
        ],
    )(idx)


def _sc_dispatch_body(idx_hbm, gates_hbm, x_hbm, counts_hbm,
                      perm_hbm, gsort_hbm, xsort_hbm, meta_hbm,
                      idx_v, g_v, cnt_v, perm_v, gsort_v, row_v, xbuf, sem):
    w = _wid()
    pltpu.async_copy(idx_hbm, idx_v, sem).wait()
    pltpu.async_copy(gates_hbm, g_v, sem).wait()
    pltpu.async_copy(counts_hbm, cnt_v, sem).wait()
    iota = lax.iota(jnp.int32, L)

    def scan_row(r, carry):
        acc, s0, s1 = carry
        crow = cnt_v[r, :]
        c0 = jnp.sum(jnp.where(iota == 0, crow, 0))
        c1 = jnp.sum(jnp.where(iota == 1, crow, 0))
        s0 = jnp.where(r == w, acc, s0)
        s1 = jnp.where(r == w, acc + _r8(c0), s1)
        return acc + _r8(c0) + _r8(c1), s0, s1

    _, s0, s1 = lax.fori_loop(0, NW, scan_row, (jnp.int32(0),) * 3)
    myrow = cnt_v[w, :]
    c0 = jnp.sum(jnp.where(iota == 0, myrow, 0))
    c1 = jnp.sum(jnp.where(iota == 1, myrow, 0))

    row = jnp.zeros((L,), jnp.int32)
    for lane, val in ((0, c0), (1, c1), (2, s0), (3, s1)):
        row = jnp.where(iota == lane, val, row)
    row_v[...] = row
    pltpu.async_copy(row_v, meta_hbm.at[w], sem).wait()

    for j, (s_j, c_j) in enumerate(((s0, c0), (s1, c1))):
        e = w * EPW + j

        def build(i, ptr):
            off = pl.multiple_of(i * L, L)
            v = idx_v[pl.ds(off, L)]
            m = v == e
            mi = m.astype(jnp.int32)
            pos = ptr + plsc.cumsum(mi) - 1
            plsc.store_scatter(perm_v, [pos], i * L + iota, mask=m)
            plsc.store_scatter(gsort_v, [pos], g_v[pl.ds(off, L)], mask=m)
            return ptr + jnp.sum(mi)

        end = lax.fori_loop(0, NV, build, s_j)
        padmask = iota < _r8(c_j) - c_j
        plsc.store_scatter(perm_v, [end + iota], jnp.full((L,), T, jnp.int32),
                           mask=padmask)
        plsc.store_scatter(gsort_v, [end + iota], jnp.zeros((L,), jnp.float32),
                           mask=padmask)

        def flush(k, carry):
            off = pl.multiple_of(s_j + k * C8, 8)
            pltpu.async_copy(perm_v.at[pl.ds(off, C8)],
                             perm_hbm.at[pl.ds(off, C8)], sem).wait()
            pltpu.async_copy(gsort_v.at[pl.ds(off, C8)],
                             gsort_hbm.at[pl.ds(off, C8)], sem).wait()
            pltpu.async_copy(x_hbm.at[perm_v.at[pl.ds(off, C8)]], xbuf,
                             sem).wait()
            pltpu.async_copy(xbuf, xsort_hbm.at[pl.ds(off, C8)], sem).wait()
            return carry

        lax.fori_loop(0, _r8(c_j) // C8, flush, 0)


def _sc_dispatch(idx, gates, x_pad, counts):
    return pl.kernel(
        _sc_dispatch_body,
        out_type=(
            jax.ShapeDtypeStruct((X_ROWS,), jnp.int32),
            jax.ShapeDtypeStruct((X_ROWS,), jnp.float32),
            jax.ShapeDtypeStruct((X_ROWS, D_MODEL), jnp.float32),
            jax.ShapeDtypeStruct((NW, L), jnp.int32),
        ),
        mesh=_sc_mesh(),
        compiler_params=pltpu.CompilerParams(needs_layout_passes=False),
        scratch_types=[
            pltpu.VMEM((T,), jnp.int32),
            pltpu.VMEM((T,), jnp.float32),
            pltpu.VMEM((NW, L), jnp.int32),
            pltpu.VMEM((X_ROWS,), jnp.int32),
            pltpu.VMEM((X_ROWS,), jnp.float32),
            pltpu.VMEM((L,), jnp.int32),
            pltpu.VMEM((C8, D_MODEL), jnp.float32),
            pltpu.SemaphoreType.DMA,
        ],
    )(idx, gates, x_pad, counts)


def _sc_combine_body(ysort_hbm, perm_hbm, meta_hbm, outp_hbm,
                     cnt_v, idxbuf, ybuf, sem):
    w = _wid()
    pltpu.async_copy(meta_hbm, cnt_v, sem).wait()
    iota = lax.iota(jnp.int32, L)
    myrow = cnt_v[w, :]
    c0 = jnp.sum(jnp.where(iota == 0, myrow, 0))
    c1 = jnp.sum(jnp.where(iota == 1, myrow, 0))
    s0 = jnp.sum(jnp.where(iota == 2, myrow, 0))
    s1 = jnp.sum(jnp.where(iota == 3, myrow, 0))

    for s_j, c_j in ((s0, c0), (s1, c1)):
        def push(k, carry):
            off = pl.multiple_of(s_j + k * C8, 8)
            pltpu.async_copy(perm_hbm.at[pl.ds(off, C8)], idxbuf, sem).wait()
            pltpu.async_copy(ysort_hbm.at[pl.ds(off, C8)], ybuf, sem).wait()
            pltpu.async_copy(ybuf, outp_hbm.at[idxbuf], sem).wait()
            return carry

        lax.fori_loop(0, _r8(c_j) // C8, push, 0)


def _sc_combine(ysort, perm, meta):
    return pl.kernel(
        _sc_combine_body,
        out_type=jax.ShapeDtypeStruct((T + 8, D_MODEL), jnp.float32),
        mesh=_sc_mesh(),
        compiler_params=pltpu.CompilerParams(needs_layout_passes=False),
        scratch_types=[
            pltpu.VMEM((NW, L), jnp.int32),
            pltpu.VMEM((C8,), jnp.int32),
            pltpu.VMEM((C8, D_MODEL), jnp.float32),
            pltpu.SemaphoreType.DMA,
        ],
    )(ysort, perm, meta)


def _ffn_body(meta_ref, x_ref, g_ref, w1_ref, w2_ref, y_ref):
    e = pl.program_id(0)
    s = meta_ref[0, e]
    n = meta_ref[1, e]
    nch = (n + BT - 1) // BT

    def chunk(i, carry):
        off = pl.multiple_of(s + i * BT, 8)
        xb = x_ref[pl.ds(off, BT), :]
        h = jax.lax.dot_general(xb, w1_ref[0], (((1,), (1,)), ((), ())),
                                preferred_element_type=jnp.float32)
        h = jax.nn.gelu(h)
        y = jax.lax.dot_general(h, w2_ref[0], (((1,), (1,)), ((), ())),
                                preferred_element_type=jnp.float32)
        y = y * g_ref[pl.ds(off, BT), :]
        y_ref[pl.ds(off, BT), :] = y
        return carry

    jax.lax.fori_loop(0, nch, chunk, 0)


def _grouped_ffn(meta, x_sorted, gates_sorted, W1, W2, interpret=False):
    grid_spec = pltpu.PrefetchScalarGridSpec(
        num_scalar_prefetch=1,
        grid=(E,),
        in_specs=[
            pl.BlockSpec((X_ROWS, D_MODEL), lambda e, m: (0, 0)),
            pl.BlockSpec((X_ROWS, 1), lambda e, m: (0, 0)),
            pl.BlockSpec((1, D_FF, D_MODEL), lambda e, m: (e, 0, 0)),
            pl.BlockSpec((1, D_MODEL, D_FF), lambda e, m: (e, 0, 0)),
        ],
        out_specs=pl.BlockSpec((X_ROWS, D_MODEL), lambda e, m: (0, 0)),
    )
    return pl.pallas_call(
        _ffn_body,
        grid_spec=grid_spec,
        out_shape=jax.ShapeDtypeStruct((X_ROWS, D_MODEL), jnp.float32),
        interpret=interpret,
    )(meta, x_sorted, gates_sorted, W1, W2)


def _dispatch_jnp(x, idx, gates):
    counts = jnp.sum(idx[:, None] == jnp.arange(E)[None, :], axis=0)
    padded = (counts + 7) & ~7
    starts = jnp.concatenate([jnp.zeros((1,), jnp.int32),
                              jnp.cumsum(padded)[:-1].astype(jnp.int32)])
    tight = jnp.concatenate([jnp.zeros((1,), jnp.int32),
                             jnp.cumsum(counts)[:-1].astype(jnp.int32)])
    order = jnp.argsort(idx)
    se = idx[order]
    dest = starts[se] + (jnp.arange(T, dtype=jnp.int32) - tight[se])
    perm = jnp.full((X_ROWS,), T, jnp.int32).at[dest].set(order)
    pos = jnp.zeros((T,), jnp.int32).at[order].set(dest)
    x_pad = jnp.concatenate([x, jnp.zeros((8, D_MODEL), x.dtype)])
    g_pad = jnp.concatenate([gates, jnp.zeros((8,), gates.dtype)])
    x_sorted = x_pad[perm]
    gates_sorted = g_pad[perm]
    meta = jnp.stack([starts, counts.astype(jnp.int32)])
    return meta, x_sorted, gates_sorted, pos


def kernel(input, expert_probs, expert_indices, W1, W2):
    x = input.reshape(-1, D_MODEL)
    idx = expert_indices.reshape(-1).astype(jnp.int32)
    gates = expert_probs.reshape(-1)
    x_pad = jnp.concatenate([x, jnp.zeros((8, D_MODEL), x.dtype)])
    counts = _sc_count(idx)
    perm, gsort, xsort, meta = _sc_dispatch(idx, gates, x_pad, counts)
    meta_tc = jnp.stack([meta[:, 2:2 + EPW].reshape(E),
                         meta[:, 0:EPW].reshape(E)])
    ysort = _grouped_ffn(meta_tc, xsort, gsort[:, None], W1, W2)
    outp = _sc_combine(ysort, perm, meta)
    return outp[:T].reshape(*input.shape[:-1], D_MODEL)


# speedup vs baseline: 1.0530x; 1.0066x over previous
import functools

import jax
import jax.numpy as jnp
from jax import lax
from jax.experimental import pallas as pl
from jax.experimental.pallas import tpu as pltpu
from jax.experimental.pallas import tpu_sc as plsc

E = 64
D_MODEL = 768
D_FF = 3072
T = 2048
BT = 64
X_ROWS = 2560
NC = 2
NS = 16
NW = NC * NS
EPW = E // NW
L = 16
NV = T // L
C8 = 8

@functools.cache
def _sc_mesh():
    return plsc.VectorSubcoreMesh(core_axis_name="c", subcore_axis_name="s")


def _wid():
    return lax.axis_index("s") * NC + lax.axis_index("c")


def _r8(n):
    return ((n + 7) >> 3) << 3


def _sc_count_body(idx_hbm, counts_hbm, idx_v, row_v, sem):
    w = _wid()
    pltpu.async_copy(idx_hbm, idx_v, sem).wait()
    iota = lax.iota(jnp.int32, L)

    def step(i, carry):
        c = list(carry)
        off = pl.multiple_of(i * L, L)
        v = idx_v[pl.ds(off, L)]
        for j in range(EPW):
            c[j] = c[j] + jnp.sum((v == w * EPW + j).astype(jnp.int32))
        return tuple(c)

    counts = lax.fori_loop(0, NV, step, (jnp.int32(0),) * EPW)
    row = jnp.zeros((L,), jnp.int32)
    for j in range(EPW):
        row = jnp.where(iota == j, counts[j], row)
    row_v[...] = row
    pltpu.async_copy(row_v, counts_hbm.at[w], sem).wait()


def _sc_count(idx):
    return pl.kernel(
        _sc_count_body,
        out_type=jax.ShapeDtypeStruct((NW, L), jnp.int32),
        mesh=_sc_mesh(),
        compiler_params=pltpu.CompilerParams(needs_layout_passes=False),
        scratch_types=[
            pltpu.VMEM((T,), jnp.int32),
            pltpu.VMEM((L,), jnp.int32),
            pltpu.SemaphoreType.DMA,
        ],
    )(idx)


def _sc_dispatch_body(idx_hbm, gates_hbm, x_hbm, counts_hbm,
                      perm_hbm, gsort_hbm, xsort_hbm, meta_hbm,
                      idx_v, g_v, cnt_v, perm_v, gsort_v, row_v, xbuf, sem):
    w = _wid()
    pltpu.async_copy(idx_hbm, idx_v, sem).wait()
    pltpu.async_copy(gates_hbm, g_v, sem).wait()
    pltpu.async_copy(counts_hbm, cnt_v, sem).wait()
    iota = lax.iota(jnp.int32, L)

    def scan_row(r, carry):
        acc, s0, s1 = carry
        crow = cnt_v[r, :]
        c0 = jnp.sum(jnp.where(iota == 0, crow, 0))
        c1 = jnp.sum(jnp.where(iota == 1, crow, 0))
        s0 = jnp.where(r == w, acc, s0)
        s1 = jnp.where(r == w, acc + _r8(c0), s1)
        return acc + _r8(c0) + _r8(c1), s0, s1

    _, s0, s1 = lax.fori_loop(0, NW, scan_row, (jnp.int32(0),) * 3)
    myrow = cnt_v[w, :]
    c0 = jnp.sum(jnp.where(iota == 0, myrow, 0))
    c1 = jnp.sum(jnp.where(iota == 1, myrow, 0))

    row = jnp.zeros((L,), jnp.int32)
    for lane, val in ((0, c0), (1, c1), (2, s0), (3, s1)):
        row = jnp.where(iota == lane, val, row)
    row_v[...] = row
    pltpu.async_copy(row_v, meta_hbm.at[w], sem).wait()

    for j, (s_j, c_j) in enumerate(((s0, c0), (s1, c1))):
        e = w * EPW + j

        def build(i, ptr):
            off = pl.multiple_of(i * L, L)
            v = idx_v[pl.ds(off, L)]
            m = v == e
            mi = m.astype(jnp.int32)
            pos = ptr + plsc.cumsum(mi) - 1
            plsc.store_scatter(perm_v, [pos], i * L + iota, mask=m)
            plsc.store_scatter(gsort_v, [pos], g_v[pl.ds(off, L)], mask=m)
            return ptr + jnp.sum(mi)

        end = lax.fori_loop(0, NV, build, s_j)
        padmask = iota < _r8(c_j) - c_j
        plsc.store_scatter(perm_v, [end + iota], jnp.full((L,), T, jnp.int32),
                           mask=padmask)
        plsc.store_scatter(gsort_v, [end + iota], jnp.zeros((L,), jnp.float32),
                           mask=padmask)

        def flush(k, carry):
            off = pl.multiple_of(s_j + k * C8, 8)
            cp_p = pltpu.make_async_copy(perm_v.at[pl.ds(off, C8)],
                                         perm_hbm.at[pl.ds(off, C8)], sem)
            cp_g = pltpu.make_async_copy(gsort_v.at[pl.ds(off, C8)],
                                         gsort_hbm.at[pl.ds(off, C8)], sem)
            cp_x = pltpu.make_async_copy(x_hbm.at[perm_v.at[pl.ds(off, C8)]],
                                         xbuf, sem)
            cp_p.start()
            cp_g.start()
            cp_x.start()
            cp_p.wait()
            cp_g.wait()
            cp_x.wait()
            pltpu.async_copy(xbuf, xsort_hbm.at[pl.ds(off, C8)], sem).wait()
            return carry

        lax.fori_loop(0, _r8(c_j) // C8, flush, 0)


def _sc_dispatch(idx, gates, x_pad, counts):
    return pl.kernel(
        _sc_dispatch_body,
        out_type=(
            jax.ShapeDtypeStruct((X_ROWS,), jnp.int32),
            jax.ShapeDtypeStruct((X_ROWS,), jnp.float32),
            jax.ShapeDtypeStruct((X_ROWS, D_MODEL), jnp.float32),
            jax.ShapeDtypeStruct((NW, L), jnp.int32),
        ),
        mesh=_sc_mesh(),
        compiler_params=pltpu.CompilerParams(needs_layout_passes=False),
        scratch_types=[
            pltpu.VMEM((T,), jnp.int32),
            pltpu.VMEM((T,), jnp.float32),
            pltpu.VMEM((NW, L), jnp.int32),
            pltpu.VMEM((X_ROWS,), jnp.int32),
            pltpu.VMEM((X_ROWS,), jnp.float32),
            pltpu.VMEM((L,), jnp.int32),
            pltpu.VMEM((C8, D_MODEL), jnp.float32),
            pltpu.SemaphoreType.DMA,
        ],
    )(idx, gates, x_pad, counts)


def _sc_combine_body(ysort_hbm, perm_hbm, meta_hbm, outp_hbm,
                     cnt_v, idxbuf, ybuf, sem):
    w = _wid()
    pltpu.async_copy(meta_hbm, cnt_v, sem).wait()
    iota = lax.iota(jnp.int32, L)
    myrow = cnt_v[w, :]
    c0 = jnp.sum(jnp.where(iota == 0, myrow, 0))
    c1 = jnp.sum(jnp.where(iota == 1, myrow, 0))
    s0 = jnp.sum(jnp.where(iota == 2, myrow, 0))
    s1 = jnp.sum(jnp.where(iota == 3, myrow, 0))

    for s_j, c_j in ((s0, c0), (s1, c1)):
        def push(k, carry):
            off = pl.multiple_of(s_j + k * C8, 8)
            cp_i = pltpu.make_async_copy(perm_hbm.at[pl.ds(off, C8)],
                                         idxbuf, sem)
            cp_y = pltpu.make_async_copy(ysort_hbm.at[pl.ds(off, C8)],
                                         ybuf, sem)
            cp_i.start()
            cp_y.start()
            cp_i.wait()
            cp_y.wait()
            pltpu.async_copy(ybuf, outp_hbm.at[idxbuf], sem).wait()
            return carry

        lax.fori_loop(0, _r8(c_j) // C8, push, 0)


def _sc_combine(ysort, perm, meta):
    return pl.kernel(
        _sc_combine_body,
        out_type=jax.ShapeDtypeStruct((T + 8, D_MODEL), jnp.float32),
        mesh=_sc_mesh(),
        compiler_params=pltpu.CompilerParams(needs_layout_passes=False),
        scratch_types=[
            pltpu.VMEM((NW, L), jnp.int32),
            pltpu.VMEM((C8,), jnp.int32),
            pltpu.VMEM((C8, D_MODEL), jnp.float32),
            pltpu.SemaphoreType.DMA,
        ],
    )(ysort, perm, meta)


def _ffn_body(meta_ref, x_ref, g_ref, w1_ref, w2_ref, y_ref):
    e = pl.program_id(0)
    s = meta_ref[0, e]
    n = meta_ref[1, e]
    nch = (n + BT - 1) // BT

    def chunk(i, carry):
        off = pl.multiple_of(s + i * BT, 8)
        xb = x_ref[pl.ds(off, BT), :]
        h = jax.lax.dot_general(xb, w1_ref[0], (((1,), (1,)), ((), ())),
                                preferred_element_type=jnp.float32)
        h = jax.nn.gelu(h)
        y = jax.lax.dot_general(h, w2_ref[0], (((1,), (1,)), ((), ())),
                                preferred_element_type=jnp.float32)
        y = y * g_ref[pl.ds(off, BT), :]
        y_ref[pl.ds(off, BT), :] = y
        return carry

    jax.lax.fori_loop(0, nch, chunk, 0)


def _grouped_ffn(meta, x_sorted, gates_sorted, W1, W2, interpret=False):
    grid_spec = pltpu.PrefetchScalarGridSpec(
        num_scalar_prefetch=1,
        grid=(E,),
        in_specs=[
            pl.BlockSpec((X_ROWS, D_MODEL), lambda e, m: (0, 0)),
            pl.BlockSpec((X_ROWS, 1), lambda e, m: (0, 0)),
            pl.BlockSpec((1, D_FF, D_MODEL), lambda e, m: (e, 0, 0)),
            pl.BlockSpec((1, D_MODEL, D_FF), lambda e, m: (e, 0, 0)),
        ],
        out_specs=pl.BlockSpec((X_ROWS, D_MODEL), lambda e, m: (0, 0)),
    )
    return pl.pallas_call(
        _ffn_body,
        grid_spec=grid_spec,
        out_shape=jax.ShapeDtypeStruct((X_ROWS, D_MODEL), jnp.float32),
        interpret=interpret,
    )(meta, x_sorted, gates_sorted, W1, W2)


def kernel(input, expert_probs, expert_indices, W1, W2):
    x = input.reshape(-1, D_MODEL)
    idx = expert_indices.reshape(-1).astype(jnp.int32)
    gates = expert_probs.reshape(-1)
    x_pad = jnp.concatenate([x, jnp.zeros((8, D_MODEL), x.dtype)])
    counts = _sc_count(idx)
    perm, gsort, xsort, meta = _sc_dispatch(idx, gates, x_pad, counts)
    meta_tc = jnp.stack([meta[:, 2:2 + EPW].reshape(E),
                         meta[:, 0:EPW].reshape(E)])
    ysort = _grouped_ffn(meta_tc, xsort, gsort[:, None], W1, W2)
    outp = _sc_combine(ysort, perm, meta)
    return outp[:T].reshape(*input.shape[:-1], D_MODEL)
